# baseline (speedup 1.0000x reference)
import functools

import jax
import jax.numpy as jnp
from jax import lax
from jax.experimental import pallas as pl
from jax.experimental.pallas import tpu as pltpu
from jax.experimental.pallas import tpu_sc as plsc

_SEQ_LEN = 200
_EMBED_DIM = 64


def _make_sc_copy():
    info = plsc.get_sparse_core_info()
    num_cores, num_subcores = info.num_cores, info.num_subcores
    num_workers = num_cores * num_subcores
    total = _SEQ_LEN * _EMBED_DIM
    per_worker = total // num_workers
    assert per_worker * num_workers == total and per_worker % 8 == 0

    mesh = plsc.VectorSubcoreMesh(core_axis_name="c", subcore_axis_name="s")

    @functools.partial(
        pl.kernel,
        mesh=mesh,
        out_type=jax.ShapeDtypeStruct((total,), jnp.float32),
        scratch_types=[pltpu.VMEM((per_worker,), jnp.float32)],
    )
    def sc_copy(table_hbm, out_hbm, buf_vmem):
        wid = lax.axis_index("s") * num_cores + lax.axis_index("c")
        base = wid * per_worker
        pltpu.sync_copy(table_hbm.at[pl.ds(_EMBED_DIM + base, per_worker)], buf_vmem)
        pltpu.sync_copy(buf_vmem, out_hbm.at[pl.ds(base, per_worker)])

    return sc_copy


_sc_copy = _make_sc_copy()


def kernel(x_item_embeddings, pos_table):
    del x_item_embeddings
    flat = pos_table.reshape(-1)
    out = _sc_copy(flat)
    return out.reshape(_SEQ_LEN, _EMBED_DIM)


# speedup vs baseline: 1.0618x; 1.0618x over previous
import functools

import jax
import jax.numpy as jnp
from jax import lax
from jax.experimental import pallas as pl
from jax.experimental.pallas import tpu as pltpu
from jax.experimental.pallas import tpu_sc as plsc

_SEQ_LEN = 200
_EMBED_DIM = 64


def _make_sc_copy():
    info = plsc.get_sparse_core_info()
    num_cores, num_subcores = 1, info.num_subcores
    num_workers = num_cores * num_subcores
    total = _SEQ_LEN * _EMBED_DIM
    per_worker = total // num_workers
    assert per_worker * num_workers == total and per_worker % 8 == 0

    mesh = plsc.VectorSubcoreMesh(
        core_axis_name="c", subcore_axis_name="s", num_cores=num_cores
    )

    @functools.partial(
        pl.kernel,
        mesh=mesh,
        out_type=jax.ShapeDtypeStruct((total,), jnp.float32),
        scratch_types=[pltpu.VMEM((per_worker,), jnp.float32)],
    )
    def sc_copy(table_hbm, out_hbm, buf_vmem):
        wid = lax.axis_index("s") * num_cores + lax.axis_index("c")
        base = wid * per_worker
        pltpu.sync_copy(table_hbm.at[pl.ds(_EMBED_DIM + base, per_worker)], buf_vmem)
        pltpu.sync_copy(buf_vmem, out_hbm.at[pl.ds(base, per_worker)])

    return sc_copy


_sc_copy = _make_sc_copy()


def kernel(x_item_embeddings, pos_table):
    del x_item_embeddings
    flat = pos_table.reshape(-1)
    out = _sc_copy(flat)
    return out.reshape(_SEQ_LEN, _EMBED_DIM)
